# baseline (speedup 1.0000x reference)
import functools

import jax
import jax.numpy as jnp
from jax import lax
from jax.experimental import pallas as pl
from jax.experimental.pallas import tpu as pltpu
from jax.experimental.pallas import tpu_sc as plsc

B, C, H, W = 256, 3, 128, 128
T = 1000
L = 16
ROWS = C * H
R = 32


def _sc_gather_body(t_hbm, sab_hbm, somab_hbm, osab_hbm, osomab_hbm,
                    idx_v, tab1_v, tab2_v, v1_v, v2_v):
    info = plsc.get_sparse_core_info()
    nc = info.num_cores
    wid = lax.axis_index("s") * nc + lax.axis_index("c")
    nchunks = B // L

    @pl.when(wid < nchunks)
    def _():
        base = wid * L
        pltpu.sync_copy(t_hbm.at[pl.ds(base, L)], idx_v)
        pltpu.sync_copy(sab_hbm, tab1_v)
        pltpu.sync_copy(somab_hbm, tab2_v)
        idx = idx_v[...]
        v1_v[...] = plsc.load_gather(tab1_v, [idx])
        v2_v[...] = plsc.load_gather(tab2_v, [idx])
        pltpu.sync_copy(v1_v, osab_hbm.at[pl.ds(base, L)])
        pltpu.sync_copy(v2_v, osomab_hbm.at[pl.ds(base, L)])


def _sc_gather(t, sab_table, somab_table):
    mesh = plsc.VectorSubcoreMesh(core_axis_name="c", subcore_axis_name="s")
    fn = functools.partial(
        pl.kernel,
        mesh=mesh,
        compiler_params=pltpu.CompilerParams(needs_layout_passes=False),
        out_type=[
            jax.ShapeDtypeStruct((B,), jnp.float32),
            jax.ShapeDtypeStruct((B,), jnp.float32),
        ],
        scratch_types=[
            pltpu.VMEM((L,), jnp.int32),
            pltpu.VMEM((T,), jnp.float32),
            pltpu.VMEM((T,), jnp.float32),
            pltpu.VMEM((L,), jnp.float32),
            pltpu.VMEM((L,), jnp.float32),
        ],
    )(_sc_gather_body)
    return fn(t, sab_table, somab_table)


def _tc_body(sab_ref, somab_ref, x_ref, n_ref, o_ref):
    i = pl.program_id(0)
    for r in range(R):
        s1 = sab_ref[i * R + r]
        s2 = somab_ref[i * R + r]
        o_ref[r] = s1 * x_ref[r] + s2 * n_ref[r]


def _tc_fma(sab_vals, somab_vals, x3, n3):
    return pl.pallas_call(
        _tc_body,
        grid=(B // R,),
        in_specs=[
            pl.BlockSpec(memory_space=pltpu.SMEM),
            pl.BlockSpec(memory_space=pltpu.SMEM),
            pl.BlockSpec((R, ROWS, W), lambda i: (i, 0, 0)),
            pl.BlockSpec((R, ROWS, W), lambda i: (i, 0, 0)),
        ],
        out_specs=pl.BlockSpec((R, ROWS, W), lambda i: (i, 0, 0)),
        out_shape=jax.ShapeDtypeStruct((B, ROWS, W), jnp.float32),
    )(sab_vals, somab_vals, x3, n3)


def kernel(x_0, t, noise, sqrt_alpha_bar, sqrt_one_minus_alpha_bar):
    t32 = t.astype(jnp.int32)
    sab_vals, somab_vals = _sc_gather(t32, sqrt_alpha_bar, sqrt_one_minus_alpha_bar)
    x3 = x_0.reshape(B, ROWS, W)
    n3 = noise.reshape(B, ROWS, W)
    x_t = _tc_fma(sab_vals, somab_vals, x3, n3)
    return x_t.reshape(B, C, H, W), noise


# speedup vs baseline: 1.1977x; 1.1977x over previous
import functools

import jax
import jax.numpy as jnp
from jax import lax
from jax.experimental import pallas as pl
from jax.experimental.pallas import tpu as pltpu
from jax.experimental.pallas import tpu_sc as plsc

B, C, H, W = 256, 3, 128, 128
T = 1000
L = 16
ROWS = C * H
R = 32


def _sc_gather_body(t_hbm, sab_hbm, somab_hbm, osab_hbm, osomab_hbm,
                    idx_v, tab1_v, tab2_v, v1_v, v2_v):
    info = plsc.get_sparse_core_info()
    nc = info.num_cores
    wid = lax.axis_index("s") * nc + lax.axis_index("c")
    nchunks = B // L

    @pl.when(wid < nchunks)
    def _():
        base = wid * L
        pltpu.sync_copy(t_hbm.at[pl.ds(base, L)], idx_v)
        pltpu.sync_copy(sab_hbm, tab1_v)
        pltpu.sync_copy(somab_hbm, tab2_v)
        idx = idx_v[...]
        v1_v[...] = plsc.load_gather(tab1_v, [idx])
        v2_v[...] = plsc.load_gather(tab2_v, [idx])
        pltpu.sync_copy(v1_v, osab_hbm.at[pl.ds(base, L)])
        pltpu.sync_copy(v2_v, osomab_hbm.at[pl.ds(base, L)])


def _sc_gather(t, sab_table, somab_table):
    mesh = plsc.VectorSubcoreMesh(core_axis_name="c", subcore_axis_name="s")
    fn = functools.partial(
        pl.kernel,
        mesh=mesh,
        compiler_params=pltpu.CompilerParams(needs_layout_passes=False),
        out_type=[
            jax.ShapeDtypeStruct((B,), jnp.float32),
            jax.ShapeDtypeStruct((B,), jnp.float32),
        ],
        scratch_types=[
            pltpu.VMEM((L,), jnp.int32),
            pltpu.VMEM((T,), jnp.float32),
            pltpu.VMEM((T,), jnp.float32),
            pltpu.VMEM((L,), jnp.float32),
            pltpu.VMEM((L,), jnp.float32),
        ],
    )(_sc_gather_body)
    return fn(t, sab_table, somab_table)


def _tc_body(sab_ref, somab_ref, x_ref, n_ref, o_ref):
    i = pl.program_id(0)
    for r in range(R):
        s1 = sab_ref[i * R + r]
        s2 = somab_ref[i * R + r]
        o_ref[r] = s1 * x_ref[r] + s2 * n_ref[r]


def _tc_fma(sab_vals, somab_vals, x3, n3):
    return pl.pallas_call(
        _tc_body,
        grid=(B // R,),
        in_specs=[
            pl.BlockSpec(memory_space=pltpu.SMEM),
            pl.BlockSpec(memory_space=pltpu.SMEM),
            pl.BlockSpec((R, ROWS, W), lambda i: (i, 0, 0)),
            pl.BlockSpec((R, ROWS, W), lambda i: (i, 0, 0)),
        ],
        out_specs=pl.BlockSpec((R, ROWS, W), lambda i: (i, 0, 0)),
        out_shape=jax.ShapeDtypeStruct((B, ROWS, W), jnp.float32),
    )(sab_vals, somab_vals, x3, n3)


def kernel(x_0, t, noise, sqrt_alpha_bar, sqrt_one_minus_alpha_bar):
    t32 = t.astype(jnp.int32)
    sab_vals = jnp.take(sqrt_alpha_bar, t32, axis=0)
    somab_vals = jnp.take(sqrt_one_minus_alpha_bar, t32, axis=0)
    x3 = x_0.reshape(B, ROWS, W)
    n3 = noise.reshape(B, ROWS, W)
    x_t = _tc_fma(sab_vals, somab_vals, x3, n3)
    return x_t.reshape(B, C, H, W), noise
